# Initial kernel scaffold; baseline (speedup 1.0000x reference)
import jax
import jax.numpy as jnp
from jax.experimental import pallas as pl


def kernel(text, unit, W_text, W_unit):
    raise NotImplementedError("write your pallas kernel here")


# speedup vs baseline: 5.8476x; 5.8476x over previous
import jax
import jax.numpy as jnp
from jax import lax
from jax.experimental import pallas as pl
from jax.experimental.pallas import tpu as pltpu
from jax.experimental.pallas import tpu_sc as plsc

NC = 2
NS = 16
NW = NC * NS

D = 128
CHUNK = 80

TEXT_ROWS = 1024 * 50
UNIT_ROWS = 1024 * 200
T_PER_W = TEXT_ROWS // NW
U_PER_W = UNIT_ROWS // NW
T_CHUNKS = T_PER_W // CHUNK
U_CHUNKS = U_PER_W // CHUNK


def _body(text_idx, unit_idx, w_text, w_unit, out_t, out_u,
          idx_t, idx_u, buf, gsem, osem):
  wid = lax.axis_index("s") * NC + lax.axis_index("c")

  pltpu.sync_copy(text_idx.at[wid], idx_t)
  pltpu.sync_copy(unit_idx.at[wid], idx_u)

  def run_table(idx_v, nchunks, table, out, base):
    pltpu.async_copy(table.at[idx_v.at[0]], buf.at[0], gsem.at[0])
    pltpu.async_copy(table.at[idx_v.at[1]], buf.at[1], gsem.at[1])

    @pl.loop(0, nchunks, step=2)
    def _(j):
      for b in range(2):
        c = j + b
        pltpu.make_async_copy(table.at[idx_v.at[c]], buf.at[b],
                              gsem.at[b]).wait()
        rows = out.at[pl.ds(base + c * CHUNK, CHUNK)]
        pltpu.async_copy(buf.at[b], rows, osem.at[b])

        @pl.when(c + 2 < nchunks)
        def _():
          pltpu.make_async_copy(buf.at[b], rows, osem.at[b]).wait()
          pltpu.async_copy(table.at[idx_v.at[c + 2]], buf.at[b],
                           gsem.at[b])

    for b in range(2):
      c = nchunks - 2 + b
      rows = out.at[pl.ds(base + c * CHUNK, CHUNK)]
      pltpu.make_async_copy(buf.at[b], rows, osem.at[b]).wait()

  run_table(idx_t, T_CHUNKS, w_text, out_t, wid * T_PER_W)
  run_table(idx_u, U_CHUNKS, w_unit, out_u, wid * U_PER_W)


@jax.jit
def kernel(text, unit, W_text, W_unit):
  text3 = text.reshape(NW, T_CHUNKS, CHUNK)
  unit3 = unit.reshape(NW, U_CHUNKS, CHUNK)

  k = pl.kernel(
      _body,
      out_type=(
          jax.ShapeDtypeStruct((TEXT_ROWS, D), jnp.float32),
          jax.ShapeDtypeStruct((UNIT_ROWS, D), jnp.float32),
      ),
      mesh=plsc.VectorSubcoreMesh(core_axis_name="c", subcore_axis_name="s"),
      scratch_types=[
          pltpu.VMEM((T_CHUNKS, CHUNK), jnp.int32),
          pltpu.VMEM((U_CHUNKS, CHUNK), jnp.int32),
          pltpu.VMEM((2, CHUNK, D), jnp.float32),
          pltpu.SemaphoreType.DMA((2,)),
          pltpu.SemaphoreType.DMA((2,)),
      ],
  )
  out_t, out_u = k(text3, unit3, W_text, W_unit)
  return (out_t.reshape(1024, 50, D), out_u.reshape(1024, 200, D))


# speedup vs baseline: 6.2108x; 1.0621x over previous
import jax
import jax.numpy as jnp
from jax import lax
from jax.experimental import pallas as pl
from jax.experimental.pallas import tpu as pltpu
from jax.experimental.pallas import tpu_sc as plsc

NC = 2
NS = 16
NW = NC * NS

D = 128
CHUNK = 80
NBUF = 4

TEXT_ROWS = 1024 * 50
UNIT_ROWS = 1024 * 200
T_PER_W = TEXT_ROWS // NW
U_PER_W = UNIT_ROWS // NW
T_CHUNKS = T_PER_W // CHUNK
U_CHUNKS = U_PER_W // CHUNK


def _body(text_idx, unit_idx, w_text, w_unit, out_t, out_u,
          idx_t, idx_u, buf, gsem, osem):
  wid = lax.axis_index("s") * NC + lax.axis_index("c")

  pltpu.sync_copy(text_idx.at[wid], idx_t)
  pltpu.sync_copy(unit_idx.at[wid], idx_u)

  def run_table(idx_v, nchunks, table, out, base):
    for b in range(NBUF):
      pltpu.async_copy(table.at[idx_v.at[b]], buf.at[b], gsem.at[b])

    @pl.loop(0, nchunks, step=NBUF)
    def _(j):
      for b in range(NBUF):
        c = j + b
        pltpu.make_async_copy(table.at[idx_v.at[c]], buf.at[b],
                              gsem.at[b]).wait()
        rows = out.at[pl.ds(base + c * CHUNK, CHUNK)]
        pltpu.async_copy(buf.at[b], rows, osem.at[b])

        @pl.when(c + NBUF < nchunks)
        def _():
          pltpu.make_async_copy(buf.at[b], rows, osem.at[b]).wait()
          pltpu.async_copy(table.at[idx_v.at[c + NBUF]], buf.at[b],
                           gsem.at[b])

    for b in range(NBUF):
      c = nchunks - NBUF + b
      rows = out.at[pl.ds(base + c * CHUNK, CHUNK)]
      pltpu.make_async_copy(buf.at[b], rows, osem.at[b]).wait()

  run_table(idx_t, T_CHUNKS, w_text, out_t, wid * T_PER_W)
  run_table(idx_u, U_CHUNKS, w_unit, out_u, wid * U_PER_W)


@jax.jit
def kernel(text, unit, W_text, W_unit):
  text3 = text.reshape(NW, T_CHUNKS, CHUNK)
  unit3 = unit.reshape(NW, U_CHUNKS, CHUNK)

  k = pl.kernel(
      _body,
      out_type=(
          jax.ShapeDtypeStruct((TEXT_ROWS, D), jnp.float32),
          jax.ShapeDtypeStruct((UNIT_ROWS, D), jnp.float32),
      ),
      mesh=plsc.VectorSubcoreMesh(core_axis_name="c", subcore_axis_name="s"),
      scratch_types=[
          pltpu.VMEM((T_CHUNKS, CHUNK), jnp.int32),
          pltpu.VMEM((U_CHUNKS, CHUNK), jnp.int32),
          pltpu.VMEM((NBUF, CHUNK, D), jnp.float32),
          pltpu.SemaphoreType.DMA((NBUF,)),
          pltpu.SemaphoreType.DMA((NBUF,)),
      ],
  )
  out_t, out_u = k(text3, unit3, W_text, W_unit)
  return (out_t.reshape(1024, 50, D), out_u.reshape(1024, 200, D))
